# baseline (speedup 1.0000x reference)
import functools

import jax
import jax.numpy as jnp
from jax import lax
from jax.experimental import pallas as pl
from jax.experimental.pallas import tpu as pltpu
from jax.experimental.pallas import tpu_sc as plsc

B = 4
S = 512
A = 8
MAXI = 32
NB = 2 * MAXI + 1

NW = 32
IPW = (B * S) // NW
CH_I = 4
NCH = IPW // CH_I
PAIRS_CH = CH_I * S // 2


def _sc_body(idx_hbm, t2_hbm, out_hbm, idxj_v, gidx_v, rows_v, sem):
    wid = lax.axis_index("s") * 2 + lax.axis_index("c")
    bb = wid // (S // IPW)
    i0 = (wid % (S // IPW)) * IPW

    pltpu.sync_copy(idx_hbm.at[bb], idxj_v)

    ev_pat = lax.iota(jnp.int32, 16) * 2

    def chunk_body(c, carry):
        i_start = i0 + c * CH_I
        for ci in range(CH_I):
            xi = plsc.load_gather(
                idxj_v, [jnp.full((16,), i_start + ci, jnp.int32)])
            for jv in range(S // 32):
                j0 = jv * 32
                ve = plsc.load_gather(idxj_v, [ev_pat + j0])
                vo = plsc.load_gather(idxj_v, [ev_pat + (j0 + 1)])
                de = jnp.clip(xi - ve, -MAXI, MAXI)
                do = jnp.clip(xi - vo, -MAXI, MAXI)
                pidx = de * NB + do + (MAXI * NB + MAXI)
                p = ci * (S // 2) + jv * 16
                gidx_v[p // 128, pl.ds(p % 128, 16)] = pidx
        handles = []
        for g in range(PAIRS_CH // 128):
            dst = rows_v.at[g // 2, pl.ds((g % 2) * 128, 128)]
            handles.append(
                pltpu.async_copy(t2_hbm.at[gidx_v.at[g]], dst, sem))
        for h in handles:
            h.wait()
        pltpu.sync_copy(rows_v, out_hbm.at[bb, pl.ds(i_start, CH_I)])
        return carry

    lax.fori_loop(0, NCH, chunk_body, 0)


def kernel(index, W, b):
    idx32 = index.astype(jnp.int32)
    T = W.T + b[None, :]
    t2 = jnp.concatenate([
        jnp.broadcast_to(T[:, None, :], (NB, NB, A)),
        jnp.broadcast_to(T[None, :, :], (NB, NB, A)),
    ], axis=-1).reshape(NB * NB, 2 * A)

    mesh = plsc.VectorSubcoreMesh(core_axis_name="c", subcore_axis_name="s")
    fn = functools.partial(
        pl.kernel,
        mesh=mesh,
        compiler_params=pltpu.CompilerParams(
            needs_layout_passes=False, use_tc_tiling_on_sc=False),
        out_type=jax.ShapeDtypeStruct((B, S, S // 2, 2 * A), jnp.float32),
        scratch_types=[
            pltpu.VMEM((S,), jnp.int32),
            pltpu.VMEM((PAIRS_CH // 128, 128), jnp.int32),
            pltpu.VMEM((CH_I, S // 2, 2 * A), jnp.float32),
            pltpu.SemaphoreType.DMA,
        ],
    )(_sc_body)
    res = fn(idx32, t2)
    return res.reshape(B, S, S, A)
